# baseline (speedup 1.0000x reference)
import functools

import jax
import jax.numpy as jnp
from jax import lax
from jax.experimental import pallas as pl
from jax.experimental.pallas import tpu as pltpu
from jax.experimental.pallas import tpu_sc as plsc

N = 2048
C = 768
H = 3072
E = 16
TILE = 256
HBLK = 256
KB = H // HBLK
NT = 24
NSLOT = NT * TILE

NC, NS = 2, 16
NW = NC * NS
BPW = N // NW


def _scatter_body(x_hbm, slot_hbm, out_hbm, idx_v, rows_v, sem):
    wid = lax.axis_index("s") * NC + lax.axis_index("c")
    base = wid * BPW
    pltpu.sync_copy(slot_hbm.at[pl.ds(base, BPW)], idx_v)
    pltpu.sync_copy(x_hbm.at[pl.ds(base, BPW)], rows_v)
    pltpu.async_copy(rows_v, out_hbm.at[idx_v], sem).wait()


def _gather_body(y_hbm, slot_hbm, out_hbm, idx_v, rows_v, sem):
    wid = lax.axis_index("s") * NC + lax.axis_index("c")
    base = wid * BPW
    pltpu.sync_copy(slot_hbm.at[pl.ds(base, BPW)], idx_v)
    pltpu.async_copy(y_hbm.at[idx_v], rows_v, sem).wait()
    pltpu.sync_copy(rows_v, out_hbm.at[pl.ds(base, BPW)])


def _sc_call(body, out_rows):
    mesh = plsc.VectorSubcoreMesh(core_axis_name="c", subcore_axis_name="s")
    return pl.kernel(
        body,
        out_type=jax.ShapeDtypeStruct((out_rows, C), jnp.float32),
        mesh=mesh,
        scratch_types=[
            pltpu.VMEM((BPW,), jnp.int32),
            pltpu.VMEM((BPW, C), jnp.float32),
            pltpu.SemaphoreType.DMA,
        ],
    )


RCH = 256


def _router_body(x_ref, wg_ref, bg_ref, slot_ref, texp_ref, tvalid_ref,
                 ranks_ref, oh_ref):
    scores = jnp.dot(x_ref[...].astype(jnp.bfloat16),
                     wg_ref[...].astype(jnp.bfloat16),
                     preferred_element_type=jnp.float32) + bg_ref[0, :][None, :]
    m = jnp.max(scores, axis=1, keepdims=True)
    iota_l = jax.lax.broadcasted_iota(jnp.int32, (N, E), 1)
    eid = jnp.min(jnp.where(scores == m, iota_l, E), axis=1, keepdims=True)
    onehot = (iota_l == eid).astype(jnp.float32)
    oh_ref[...] = onehot.astype(jnp.bfloat16)

    lt = (jax.lax.broadcasted_iota(jnp.int32, (RCH, RCH), 1)
          <= jax.lax.broadcasted_iota(jnp.int32, (RCH, RCH), 0))
    lt = lt.astype(jnp.bfloat16)

    def chunk(c, running):
        row0 = pl.multiple_of(c * RCH, RCH)
        oh_c = oh_ref[pl.ds(row0, RCH), :]
        local = jax.lax.dot_general(lt, oh_c, (((1,), (0,)), ((), ())),
                                    preferred_element_type=jnp.float32)
        ranks_ref[pl.ds(row0, RCH), :] = local + running
        return running + local[RCH - 1:RCH, :]

    counts = jax.lax.fori_loop(0, N // RCH, chunk,
                               jnp.zeros((1, E), jnp.float32))

    rank_own = jnp.sum(onehot * ranks_ref[...], axis=1, keepdims=True) - 1.0

    nt = jnp.floor((counts + (TILE - 1)) * (1.0 / TILE))
    ut = (jax.lax.broadcasted_iota(jnp.int32, (E, E), 0)
          < jax.lax.broadcasted_iota(jnp.int32, (E, E), 1)).astype(jnp.bfloat16)
    nt8 = jnp.broadcast_to(nt.astype(jnp.bfloat16), (8, E))
    ts = jax.lax.dot_general(nt8, ut, (((1,), (0,)), ((), ())),
                             preferred_element_type=jnp.float32)[0:1, :]
    t_act = jnp.sum(nt, axis=1, keepdims=True)

    slot = jnp.sum(onehot * ts, axis=1, keepdims=True) * TILE + rank_own
    slot_ref[...] = slot.astype(jnp.int32)

    t_row = jax.lax.broadcasted_iota(jnp.int32, (NT, E), 0).astype(jnp.float32)
    tsb = jnp.broadcast_to(ts, (NT, E))
    ntb = jnp.broadcast_to(nt, (NT, E))
    in_e = ((t_row >= tsb) & (t_row < tsb + ntb)).astype(jnp.float32)
    lane_e = jax.lax.broadcasted_iota(jnp.int32, (NT, E), 1).astype(jnp.float32)
    texp = jnp.sum(in_e * lane_e, axis=1, keepdims=True)
    t_col = jax.lax.broadcasted_iota(jnp.int32, (NT, 1), 0).astype(jnp.float32)
    t_act_b = jnp.broadcast_to(t_act, (NT, 1))
    valid = t_col < t_act_b
    last_e = jnp.sum(jnp.where(t_col == t_act_b - 1.0, texp, 0.0),
                     axis=0, keepdims=True)
    texp = jnp.where(valid, texp, jnp.broadcast_to(last_e, (NT, 1)))
    texp_ref[...] = texp.astype(jnp.int32)
    tvalid_ref[...] = valid.astype(jnp.int32)


_router_call = pl.pallas_call(
    _router_body,
    out_shape=[
        jax.ShapeDtypeStruct((N, 1), jnp.int32),
        jax.ShapeDtypeStruct((NT, 1), jnp.int32),
        jax.ShapeDtypeStruct((NT, 1), jnp.int32),
    ],
    scratch_shapes=[
        pltpu.VMEM((N, E), jnp.float32),
        pltpu.VMEM((N, E), jnp.bfloat16),
    ],
)


H2 = H // 2


def _ffn_body(texp_ref, tvalid_ref, x_ref, w1a_ref, w1b_ref, b1_ref,
              w2a_ref, w2b_ref, b2_ref, out_ref, h_ref):
    t = pl.program_id(0)

    @pl.when(tvalid_ref[t, 0] != 0)
    def _():
        xb = x_ref[...].astype(jnp.bfloat16)
        xwa = jnp.dot(xb, w1a_ref[0].astype(jnp.bfloat16),
                      preferred_element_type=jnp.float32)
        h_ref[:, :H2] = jnp.maximum(
            xwa + b1_ref[0, 0, :H2][None, :], 0.0).astype(jnp.bfloat16)
        xwb = jnp.dot(xb, w1b_ref[0].astype(jnp.bfloat16),
                      preferred_element_type=jnp.float32)
        h_ref[:, H2:] = jnp.maximum(
            xwb + b1_ref[0, 0, H2:][None, :], 0.0).astype(jnp.bfloat16)
        ya = jnp.dot(h_ref[:, :H2], w2a_ref[0].astype(jnp.bfloat16),
                     preferred_element_type=jnp.float32)
        yb = jnp.dot(h_ref[:, H2:], w2b_ref[0].astype(jnp.bfloat16),
                     preferred_element_type=jnp.float32)
        out_ref[...] = ya + yb + b2_ref[0, 0, :][None, :]


_ffn_call = pl.pallas_call(
    _ffn_body,
    grid_spec=pltpu.PrefetchScalarGridSpec(
        num_scalar_prefetch=2,
        grid=(NT,),
        in_specs=[
            pl.BlockSpec((TILE, C), lambda t, te, tv: (t, 0)),
            pl.BlockSpec((1, C, H2), lambda t, te, tv: (te[t, 0], 0, 0)),
            pl.BlockSpec((1, C, H2), lambda t, te, tv: (te[t, 0], 0, 1)),
            pl.BlockSpec((1, 1, H), lambda t, te, tv: (te[t, 0], 0, 0)),
            pl.BlockSpec((1, H2, C), lambda t, te, tv: (te[t, 0], 0, 0)),
            pl.BlockSpec((1, H2, C), lambda t, te, tv: (te[t, 0], 1, 0)),
            pl.BlockSpec((1, 1, C), lambda t, te, tv: (te[t, 0], 0, 0)),
        ],
        out_specs=pl.BlockSpec((TILE, C), lambda t, te, tv: (t, 0)),
        scratch_shapes=[pltpu.VMEM((TILE, H), jnp.bfloat16)],
    ),
    out_shape=jax.ShapeDtypeStruct((NSLOT, C), jnp.float32),
)


def kernel(x, Wg, bg, W1, b1, W2, b2):
    Bv, Tv, _ = x.shape
    x_flat = x.reshape(Bv * Tv, C)

    slot2, texp, tile_valid = _router_call(x_flat, Wg, bg.reshape(1, E))
    slot = slot2.reshape(N)

    x_sorted = jnp.zeros((NSLOT, C), jnp.float32).at[slot].set(x_flat)
    y_sorted = _ffn_call(texp, tile_valid, x_sorted, W1, W1,
                         b1.reshape(E, 1, H), W2, W2, b2.reshape(E, 1, C))
    out = y_sorted[slot]
    return out.reshape(Bv, Tv, C)


# speedup vs baseline: 1.0082x; 1.0082x over previous
import functools

import jax
import jax.numpy as jnp
from jax import lax
from jax.experimental import pallas as pl
from jax.experimental.pallas import tpu as pltpu
from jax.experimental.pallas import tpu_sc as plsc

N = 2048
C = 768
H = 3072
E = 16
TILE = 128
NT = 31
NSLOT = NT * TILE

NC, NS = 2, 16
NW = NC * NS
BPW = N // NW


def _scatter_body(x_hbm, slot_hbm, out_hbm, idx_v, rows_v, sem):
    wid = lax.axis_index("s") * NC + lax.axis_index("c")
    base = wid * BPW
    pltpu.sync_copy(slot_hbm.at[pl.ds(base, BPW)], idx_v)
    pltpu.sync_copy(x_hbm.at[pl.ds(base, BPW)], rows_v)
    pltpu.async_copy(rows_v, out_hbm.at[idx_v], sem).wait()


def _gather_body(y_hbm, slot_hbm, out_hbm, idx_v, rows_v, sem):
    wid = lax.axis_index("s") * NC + lax.axis_index("c")
    base = wid * BPW
    pltpu.sync_copy(slot_hbm.at[pl.ds(base, BPW)], idx_v)
    pltpu.async_copy(y_hbm.at[idx_v], rows_v, sem).wait()
    pltpu.sync_copy(rows_v, out_hbm.at[pl.ds(base, BPW)])


def _sc_call(body, out_rows):
    mesh = plsc.VectorSubcoreMesh(core_axis_name="c", subcore_axis_name="s")
    return pl.kernel(
        body,
        out_type=jax.ShapeDtypeStruct((out_rows, C), jnp.float32),
        mesh=mesh,
        scratch_types=[
            pltpu.VMEM((BPW,), jnp.int32),
            pltpu.VMEM((BPW, C), jnp.float32),
            pltpu.SemaphoreType.DMA,
        ],
    )


RCH = 256


def _router_body(x_ref, wg_ref, bg_ref, slot_ref, texp_ref, tvalid_ref,
                 ranks_ref, oh_ref):
    scores = jnp.dot(x_ref[...].astype(jnp.bfloat16),
                     wg_ref[...].astype(jnp.bfloat16),
                     preferred_element_type=jnp.float32) + bg_ref[0, :][None, :]
    m = jnp.max(scores, axis=1, keepdims=True)
    iota_l = jax.lax.broadcasted_iota(jnp.int32, (N, E), 1)
    eid = jnp.min(jnp.where(scores == m, iota_l, E), axis=1, keepdims=True)
    onehot = (iota_l == eid).astype(jnp.float32)
    oh_ref[...] = onehot.astype(jnp.bfloat16)

    lt = (jax.lax.broadcasted_iota(jnp.int32, (RCH, RCH), 1)
          <= jax.lax.broadcasted_iota(jnp.int32, (RCH, RCH), 0))
    lt = lt.astype(jnp.bfloat16)

    def chunk(c, running):
        row0 = pl.multiple_of(c * RCH, RCH)
        oh_c = oh_ref[pl.ds(row0, RCH), :]
        local = jax.lax.dot_general(lt, oh_c, (((1,), (0,)), ((), ())),
                                    preferred_element_type=jnp.float32)
        ranks_ref[pl.ds(row0, RCH), :] = local + running
        return running + local[RCH - 1:RCH, :]

    counts = jax.lax.fori_loop(0, N // RCH, chunk,
                               jnp.zeros((1, E), jnp.float32))

    rank_own = jnp.sum(onehot * ranks_ref[...], axis=1, keepdims=True) - 1.0

    nt = jnp.floor((counts + (TILE - 1)) * (1.0 / TILE))
    ut = (jax.lax.broadcasted_iota(jnp.int32, (E, E), 0)
          < jax.lax.broadcasted_iota(jnp.int32, (E, E), 1)).astype(jnp.bfloat16)
    nt8 = jnp.broadcast_to(nt.astype(jnp.bfloat16), (8, E))
    ts = jax.lax.dot_general(nt8, ut, (((1,), (0,)), ((), ())),
                             preferred_element_type=jnp.float32)[0:1, :]
    t_act = jnp.sum(nt, axis=1, keepdims=True)

    slot = jnp.sum(onehot * ts, axis=1, keepdims=True) * TILE + rank_own
    slot_ref[...] = slot.astype(jnp.int32)

    t_row = jax.lax.broadcasted_iota(jnp.int32, (NT, E), 0).astype(jnp.float32)
    tsb = jnp.broadcast_to(ts, (NT, E))
    ntb = jnp.broadcast_to(nt, (NT, E))
    in_e = ((t_row >= tsb) & (t_row < tsb + ntb)).astype(jnp.float32)
    lane_e = jax.lax.broadcasted_iota(jnp.int32, (NT, E), 1).astype(jnp.float32)
    texp = jnp.sum(in_e * lane_e, axis=1, keepdims=True)
    t_col = jax.lax.broadcasted_iota(jnp.int32, (NT, 1), 0).astype(jnp.float32)
    t_act_b = jnp.broadcast_to(t_act, (NT, 1))
    valid = t_col < t_act_b
    last_e = jnp.sum(jnp.where(t_col == t_act_b - 1.0, texp, 0.0),
                     axis=0, keepdims=True)
    texp = jnp.where(valid, texp, jnp.broadcast_to(last_e, (NT, 1)))
    texp_ref[...] = texp.astype(jnp.int32)
    tvalid_ref[...] = valid.astype(jnp.int32)


_router_call = pl.pallas_call(
    _router_body,
    out_shape=[
        jax.ShapeDtypeStruct((N, 1), jnp.int32),
        jax.ShapeDtypeStruct((NT, 1), jnp.int32),
        jax.ShapeDtypeStruct((NT, 1), jnp.int32),
    ],
    scratch_shapes=[
        pltpu.VMEM((N, E), jnp.float32),
        pltpu.VMEM((N, E), jnp.bfloat16),
    ],
)


def _ffn_body(texp_ref, tvalid_ref, x_ref, w1_ref, b1_ref, w2_ref, b2_ref,
              out_ref, h_ref):
    t = pl.program_id(0)

    @pl.when(tvalid_ref[t, 0] != 0)
    def _():
        xb = x_ref[...].astype(jnp.bfloat16)
        xw = jnp.dot(xb, w1_ref[0].astype(jnp.bfloat16),
                     preferred_element_type=jnp.float32)
        h_ref[...] = jnp.maximum(xw + b1_ref[0, 0, :][None, :],
                                 0.0).astype(jnp.bfloat16)
        yk = jnp.dot(h_ref[...], w2_ref[0].astype(jnp.bfloat16),
                     preferred_element_type=jnp.float32)
        out_ref[...] = yk + b2_ref[0, 0, :][None, :]


_ffn_call = pl.pallas_call(
    _ffn_body,
    grid_spec=pltpu.PrefetchScalarGridSpec(
        num_scalar_prefetch=2,
        grid=(NT,),
        in_specs=[
            pl.BlockSpec((TILE, C), lambda t, te, tv: (t, 0)),
            pl.BlockSpec((1, C, H), lambda t, te, tv: (te[t, 0], 0, 0)),
            pl.BlockSpec((1, 1, H), lambda t, te, tv: (te[t, 0], 0, 0)),
            pl.BlockSpec((1, H, C), lambda t, te, tv: (te[t, 0], 0, 0)),
            pl.BlockSpec((1, 1, C), lambda t, te, tv: (te[t, 0], 0, 0)),
        ],
        out_specs=pl.BlockSpec((TILE, C), lambda t, te, tv: (t, 0)),
        scratch_shapes=[pltpu.VMEM((TILE, H), jnp.bfloat16)],
    ),
    out_shape=jax.ShapeDtypeStruct((NSLOT, C), jnp.float32),
)


def kernel(x, Wg, bg, W1, b1, W2, b2):
    Bv, Tv, _ = x.shape
    x_flat = x.reshape(Bv * Tv, C)

    slot2, texp, tile_valid = _router_call(x_flat, Wg, bg.reshape(1, E))
    slot = slot2.reshape(N)

    x_sorted = _sc_call(_scatter_body, NSLOT)(x_flat, slot)
    y_sorted = _ffn_call(texp, tile_valid, x_sorted, W1,
                         b1.reshape(E, 1, H), W2, b2.reshape(E, 1, C))
    out = _sc_call(_gather_body, N)(y_sorted, slot)
    return out.reshape(Bv, Tv, C)


# speedup vs baseline: 1.1145x; 1.1054x over previous
import functools

import jax
import jax.numpy as jnp
from jax import lax
from jax.experimental import pallas as pl
from jax.experimental.pallas import tpu as pltpu
from jax.experimental.pallas import tpu_sc as plsc

N = 2048
C = 768
H = 3072
E = 16
TILE = 256
NT = 24
NSLOT = NT * TILE

NC, NS = 2, 16
NW = NC * NS
BPW = N // NW


def _scatter_body(x_hbm, slot_hbm, out_hbm, idx_v, rows_v, sem):
    wid = lax.axis_index("s") * NC + lax.axis_index("c")
    base = wid * BPW
    pltpu.sync_copy(slot_hbm.at[pl.ds(base, BPW)], idx_v)
    pltpu.sync_copy(x_hbm.at[pl.ds(base, BPW)], rows_v)
    pltpu.async_copy(rows_v, out_hbm.at[idx_v], sem).wait()


def _gather_body(y_hbm, slot_hbm, out_hbm, idx_v, rows_v, sem):
    wid = lax.axis_index("s") * NC + lax.axis_index("c")
    base = wid * BPW
    pltpu.sync_copy(slot_hbm.at[pl.ds(base, BPW)], idx_v)
    pltpu.async_copy(y_hbm.at[idx_v], rows_v, sem).wait()
    pltpu.sync_copy(rows_v, out_hbm.at[pl.ds(base, BPW)])


def _sc_call(body, out_rows):
    mesh = plsc.VectorSubcoreMesh(core_axis_name="c", subcore_axis_name="s")
    return pl.kernel(
        body,
        out_type=jax.ShapeDtypeStruct((out_rows, C), jnp.float32),
        mesh=mesh,
        scratch_types=[
            pltpu.VMEM((BPW,), jnp.int32),
            pltpu.VMEM((BPW, C), jnp.float32),
            pltpu.SemaphoreType.DMA,
        ],
    )


RCH = 256


def _router_body(x_ref, wg_ref, bg_ref, slot_ref, texp_ref, tvalid_ref,
                 ranks_ref, oh_ref):
    scores = jnp.dot(x_ref[...].astype(jnp.bfloat16),
                     wg_ref[...].astype(jnp.bfloat16),
                     preferred_element_type=jnp.float32) + bg_ref[0, :][None, :]
    m = jnp.max(scores, axis=1, keepdims=True)
    iota_l = jax.lax.broadcasted_iota(jnp.int32, (N, E), 1)
    eid = jnp.min(jnp.where(scores == m, iota_l, E), axis=1, keepdims=True)
    onehot = (iota_l == eid).astype(jnp.float32)
    oh_ref[...] = onehot.astype(jnp.bfloat16)

    lt = (jax.lax.broadcasted_iota(jnp.int32, (RCH, RCH), 1)
          <= jax.lax.broadcasted_iota(jnp.int32, (RCH, RCH), 0))
    lt = lt.astype(jnp.bfloat16)

    def chunk(c, running):
        row0 = pl.multiple_of(c * RCH, RCH)
        oh_c = oh_ref[pl.ds(row0, RCH), :]
        local = jax.lax.dot_general(lt, oh_c, (((1,), (0,)), ((), ())),
                                    preferred_element_type=jnp.float32)
        ranks_ref[pl.ds(row0, RCH), :] = local + running
        return running + local[RCH - 1:RCH, :]

    counts = jax.lax.fori_loop(0, N // RCH, chunk,
                               jnp.zeros((1, E), jnp.float32))

    rank_own = jnp.sum(onehot * ranks_ref[...], axis=1, keepdims=True) - 1.0

    nt = jnp.floor((counts + (TILE - 1)) * (1.0 / TILE))
    ut = (jax.lax.broadcasted_iota(jnp.int32, (E, E), 0)
          < jax.lax.broadcasted_iota(jnp.int32, (E, E), 1)).astype(jnp.bfloat16)
    nt8 = jnp.broadcast_to(nt.astype(jnp.bfloat16), (8, E))
    ts = jax.lax.dot_general(nt8, ut, (((1,), (0,)), ((), ())),
                             preferred_element_type=jnp.float32)[0:1, :]
    t_act = jnp.sum(nt, axis=1, keepdims=True)

    slot = jnp.sum(onehot * ts, axis=1, keepdims=True) * TILE + rank_own
    slot_ref[...] = slot.astype(jnp.int32)

    t_row = jax.lax.broadcasted_iota(jnp.int32, (NT, E), 0).astype(jnp.float32)
    tsb = jnp.broadcast_to(ts, (NT, E))
    ntb = jnp.broadcast_to(nt, (NT, E))
    in_e = ((t_row >= tsb) & (t_row < tsb + ntb)).astype(jnp.float32)
    lane_e = jax.lax.broadcasted_iota(jnp.int32, (NT, E), 1).astype(jnp.float32)
    texp = jnp.sum(in_e * lane_e, axis=1, keepdims=True)
    t_col = jax.lax.broadcasted_iota(jnp.int32, (NT, 1), 0).astype(jnp.float32)
    t_act_b = jnp.broadcast_to(t_act, (NT, 1))
    valid = t_col < t_act_b
    last_e = jnp.sum(jnp.where(t_col == t_act_b - 1.0, texp, 0.0),
                     axis=0, keepdims=True)
    texp = jnp.where(valid, texp, jnp.broadcast_to(last_e, (NT, 1)))
    texp_ref[...] = texp.astype(jnp.int32)
    tvalid_ref[...] = valid.astype(jnp.int32)


_router_call = pl.pallas_call(
    _router_body,
    out_shape=[
        jax.ShapeDtypeStruct((N, 1), jnp.int32),
        jax.ShapeDtypeStruct((NT, 1), jnp.int32),
        jax.ShapeDtypeStruct((NT, 1), jnp.int32),
    ],
    scratch_shapes=[
        pltpu.VMEM((N, E), jnp.float32),
        pltpu.VMEM((N, E), jnp.bfloat16),
    ],
)


def _ffn_body(texp_ref, tvalid_ref, x_ref, w1_ref, b1_ref, w2_ref, b2_ref,
              out_ref, h_ref):
    t = pl.program_id(0)

    @pl.when(tvalid_ref[t, 0] != 0)
    def _():
        xb = x_ref[...].astype(jnp.bfloat16)
        xw = jnp.dot(xb, w1_ref[0].astype(jnp.bfloat16),
                     preferred_element_type=jnp.float32)
        h_ref[...] = jnp.maximum(xw + b1_ref[0, 0, :][None, :],
                                 0.0).astype(jnp.bfloat16)
        yk = jnp.dot(h_ref[...], w2_ref[0].astype(jnp.bfloat16),
                     preferred_element_type=jnp.float32)
        out_ref[...] = yk + b2_ref[0, 0, :][None, :]


_ffn_call = pl.pallas_call(
    _ffn_body,
    grid_spec=pltpu.PrefetchScalarGridSpec(
        num_scalar_prefetch=2,
        grid=(NT,),
        in_specs=[
            pl.BlockSpec((TILE, C), lambda t, te, tv: (t, 0)),
            pl.BlockSpec((1, C, H), lambda t, te, tv: (te[t, 0], 0, 0)),
            pl.BlockSpec((1, 1, H), lambda t, te, tv: (te[t, 0], 0, 0)),
            pl.BlockSpec((1, H, C), lambda t, te, tv: (te[t, 0], 0, 0)),
            pl.BlockSpec((1, 1, C), lambda t, te, tv: (te[t, 0], 0, 0)),
        ],
        out_specs=pl.BlockSpec((TILE, C), lambda t, te, tv: (t, 0)),
        scratch_shapes=[pltpu.VMEM((TILE, H), jnp.bfloat16)],
    ),
    out_shape=jax.ShapeDtypeStruct((NSLOT, C), jnp.float32),
)


def kernel(x, Wg, bg, W1, b1, W2, b2):
    Bv, Tv, _ = x.shape
    x_flat = x.reshape(Bv * Tv, C)

    slot2, texp, tile_valid = _router_call(x_flat, Wg, bg.reshape(1, E))
    slot = slot2.reshape(N)

    x_sorted = _sc_call(_scatter_body, NSLOT)(x_flat, slot)
    y_sorted = _ffn_call(texp, tile_valid, x_sorted, W1,
                         b1.reshape(E, 1, H), W2, b2.reshape(E, 1, C))
    out = _sc_call(_gather_body, N)(y_sorted, slot)
    return out.reshape(Bv, Tv, C)


# speedup vs baseline: 1.1159x; 1.0012x over previous
import functools

import jax
import jax.numpy as jnp
from jax import lax
from jax.experimental import pallas as pl
from jax.experimental.pallas import tpu as pltpu
from jax.experimental.pallas import tpu_sc as plsc

N = 2048
C = 768
H = 3072
E = 16
TILE = 256
NT = 24
NSLOT = NT * TILE

NC, NS = 2, 16
NW = NC * NS
BPW = N // NW


HB = BPW // 2


def _scatter_body(x_hbm, slot_hbm, out_hbm, idx_v, rows_v, sem_i, sem_a,
                  sem_b, sem_o):
    wid = lax.axis_index("s") * NC + lax.axis_index("c")
    base = wid * BPW
    cp_i = pltpu.async_copy(slot_hbm.at[pl.ds(wid * 2, 2)], idx_v, sem_i)
    cp_a = pltpu.async_copy(x_hbm.at[pl.ds(base, HB)],
                            rows_v.at[pl.ds(0, HB)], sem_a)
    cp_b = pltpu.async_copy(x_hbm.at[pl.ds(base + HB, HB)],
                            rows_v.at[pl.ds(HB, HB)], sem_b)
    cp_i.wait()
    cp_a.wait()
    s_a = pltpu.async_copy(rows_v.at[pl.ds(0, HB)],
                           out_hbm.at[idx_v.at[0]], sem_o)
    cp_b.wait()
    s_b = pltpu.async_copy(rows_v.at[pl.ds(HB, HB)],
                           out_hbm.at[idx_v.at[1]], sem_o)
    s_a.wait()
    s_b.wait()


def _gather_body(y_hbm, slot_hbm, out_hbm, idx_v, rows_v, sem_i, sem_a,
                 sem_b, sem_o):
    wid = lax.axis_index("s") * NC + lax.axis_index("c")
    base = wid * BPW
    pltpu.sync_copy(slot_hbm.at[pl.ds(wid * 2, 2)], idx_v)
    g_a = pltpu.async_copy(y_hbm.at[idx_v.at[0]],
                           rows_v.at[pl.ds(0, HB)], sem_a)
    g_b = pltpu.async_copy(y_hbm.at[idx_v.at[1]],
                           rows_v.at[pl.ds(HB, HB)], sem_b)
    g_a.wait()
    o_a = pltpu.async_copy(rows_v.at[pl.ds(0, HB)],
                           out_hbm.at[pl.ds(base, HB)], sem_o)
    g_b.wait()
    o_b = pltpu.async_copy(rows_v.at[pl.ds(HB, HB)],
                           out_hbm.at[pl.ds(base + HB, HB)], sem_o)
    o_a.wait()
    o_b.wait()


def _sc_call(body, out_rows):
    mesh = plsc.VectorSubcoreMesh(core_axis_name="c", subcore_axis_name="s")
    return pl.kernel(
        body,
        out_type=jax.ShapeDtypeStruct((out_rows, C), jnp.float32),
        mesh=mesh,
        scratch_types=[
            pltpu.VMEM((2, HB), jnp.int32),
            pltpu.VMEM((BPW, C), jnp.float32),
            pltpu.SemaphoreType.DMA,
            pltpu.SemaphoreType.DMA,
            pltpu.SemaphoreType.DMA,
            pltpu.SemaphoreType.DMA,
        ],
    )


RCH = 256


def _router_body(x_ref, wg_ref, bg_ref, slot_ref, texp_ref, tvalid_ref,
                 ranks_ref, oh_ref):
    scores = jnp.dot(x_ref[...].astype(jnp.bfloat16),
                     wg_ref[...].astype(jnp.bfloat16),
                     preferred_element_type=jnp.float32) + bg_ref[0, :][None, :]
    m = jnp.max(scores, axis=1, keepdims=True)
    iota_l = jax.lax.broadcasted_iota(jnp.int32, (N, E), 1)
    eid = jnp.min(jnp.where(scores == m, iota_l, E), axis=1, keepdims=True)
    onehot = (iota_l == eid).astype(jnp.float32)
    oh_ref[...] = onehot.astype(jnp.bfloat16)

    lt = (jax.lax.broadcasted_iota(jnp.int32, (RCH, RCH), 1)
          <= jax.lax.broadcasted_iota(jnp.int32, (RCH, RCH), 0))
    lt = lt.astype(jnp.bfloat16)

    def chunk(c, running):
        row0 = pl.multiple_of(c * RCH, RCH)
        oh_c = oh_ref[pl.ds(row0, RCH), :]
        local = jax.lax.dot_general(lt, oh_c, (((1,), (0,)), ((), ())),
                                    preferred_element_type=jnp.float32)
        ranks_ref[pl.ds(row0, RCH), :] = local + running
        return running + local[RCH - 1:RCH, :]

    counts = jax.lax.fori_loop(0, N // RCH, chunk,
                               jnp.zeros((1, E), jnp.float32))

    rank_own = jnp.sum(onehot * ranks_ref[...], axis=1, keepdims=True) - 1.0

    nt = jnp.floor((counts + (TILE - 1)) * (1.0 / TILE))
    ut = (jax.lax.broadcasted_iota(jnp.int32, (E, E), 0)
          < jax.lax.broadcasted_iota(jnp.int32, (E, E), 1)).astype(jnp.bfloat16)
    nt8 = jnp.broadcast_to(nt.astype(jnp.bfloat16), (8, E))
    ts = jax.lax.dot_general(nt8, ut, (((1,), (0,)), ((), ())),
                             preferred_element_type=jnp.float32)[0:1, :]
    t_act = jnp.sum(nt, axis=1, keepdims=True)

    slot = jnp.sum(onehot * ts, axis=1, keepdims=True) * TILE + rank_own
    slot_ref[...] = slot.astype(jnp.int32)

    t_row = jax.lax.broadcasted_iota(jnp.int32, (NT, E), 0).astype(jnp.float32)
    tsb = jnp.broadcast_to(ts, (NT, E))
    ntb = jnp.broadcast_to(nt, (NT, E))
    in_e = ((t_row >= tsb) & (t_row < tsb + ntb)).astype(jnp.float32)
    lane_e = jax.lax.broadcasted_iota(jnp.int32, (NT, E), 1).astype(jnp.float32)
    texp = jnp.sum(in_e * lane_e, axis=1, keepdims=True)
    t_col = jax.lax.broadcasted_iota(jnp.int32, (NT, 1), 0).astype(jnp.float32)
    t_act_b = jnp.broadcast_to(t_act, (NT, 1))
    valid = t_col < t_act_b
    last_e = jnp.sum(jnp.where(t_col == t_act_b - 1.0, texp, 0.0),
                     axis=0, keepdims=True)
    texp = jnp.where(valid, texp, jnp.broadcast_to(last_e, (NT, 1)))
    texp_ref[...] = texp.astype(jnp.int32)
    tvalid_ref[...] = valid.astype(jnp.int32)


_router_call = pl.pallas_call(
    _router_body,
    out_shape=[
        jax.ShapeDtypeStruct((N, 1), jnp.int32),
        jax.ShapeDtypeStruct((NT, 1), jnp.int32),
        jax.ShapeDtypeStruct((NT, 1), jnp.int32),
    ],
    scratch_shapes=[
        pltpu.VMEM((N, E), jnp.float32),
        pltpu.VMEM((N, E), jnp.bfloat16),
    ],
)


def _ffn_body(texp_ref, tvalid_ref, x_ref, w1_ref, b1_ref, w2_ref, b2_ref,
              out_ref, h_ref):
    t = pl.program_id(0)

    @pl.when(tvalid_ref[t, 0] != 0)
    def _():
        xb = x_ref[...].astype(jnp.bfloat16)
        xw = jnp.dot(xb, w1_ref[0].astype(jnp.bfloat16),
                     preferred_element_type=jnp.float32)
        h_ref[...] = jnp.maximum(xw + b1_ref[0, 0, :][None, :],
                                 0.0).astype(jnp.bfloat16)
        yk = jnp.dot(h_ref[...], w2_ref[0].astype(jnp.bfloat16),
                     preferred_element_type=jnp.float32)
        out_ref[...] = yk + b2_ref[0, 0, :][None, :]


_ffn_call = pl.pallas_call(
    _ffn_body,
    grid_spec=pltpu.PrefetchScalarGridSpec(
        num_scalar_prefetch=2,
        grid=(NT,),
        in_specs=[
            pl.BlockSpec((TILE, C), lambda t, te, tv: (t, 0)),
            pl.BlockSpec((1, C, H), lambda t, te, tv: (te[t, 0], 0, 0)),
            pl.BlockSpec((1, 1, H), lambda t, te, tv: (te[t, 0], 0, 0)),
            pl.BlockSpec((1, H, C), lambda t, te, tv: (te[t, 0], 0, 0)),
            pl.BlockSpec((1, 1, C), lambda t, te, tv: (te[t, 0], 0, 0)),
        ],
        out_specs=pl.BlockSpec((TILE, C), lambda t, te, tv: (t, 0)),
        scratch_shapes=[pltpu.VMEM((TILE, H), jnp.bfloat16)],
    ),
    out_shape=jax.ShapeDtypeStruct((NSLOT, C), jnp.float32),
)


def kernel(x, Wg, bg, W1, b1, W2, b2):
    Bv, Tv, _ = x.shape
    x_flat = x.reshape(Bv * Tv, C)

    slot2, texp, tile_valid = _router_call(x_flat, Wg, bg.reshape(1, E))
    slot = slot2.reshape(N)

    slot_2d = slot.reshape(NW * 2, HB)
    x_sorted = _sc_call(_scatter_body, NSLOT)(x_flat, slot_2d)
    y_sorted = _ffn_call(texp, tile_valid, x_sorted, W1,
                         b1.reshape(E, 1, H), W2, b2.reshape(E, 1, C))
    out = _sc_call(_gather_body, N)(y_sorted, slot_2d)
    return out.reshape(Bv, Tv, C)


# speedup vs baseline: 1.1597x; 1.0393x over previous
import functools

import jax
import jax.numpy as jnp
from jax import lax
from jax.experimental import pallas as pl
from jax.experimental.pallas import tpu as pltpu
from jax.experimental.pallas import tpu_sc as plsc

N = 2048
C = 768
H = 3072
E = 16
TILE = 256
NT = 24
NSLOT = NT * TILE

NC, NS = 2, 16
NW = NC * NS
BPW = N // NW


def _scatter_body(x_hbm, slot_hbm, out_hbm, idx_v, rows_v, sem):
    wid = lax.axis_index("s") * NC + lax.axis_index("c")
    base = wid * BPW
    pltpu.sync_copy(slot_hbm.at[pl.ds(base, BPW)], idx_v)
    pltpu.sync_copy(x_hbm.at[pl.ds(base, BPW)], rows_v)
    pltpu.async_copy(rows_v, out_hbm.at[idx_v], sem).wait()


def _gather_body(y_hbm, slot_hbm, out_hbm, idx_v, rows_v, sem):
    wid = lax.axis_index("s") * NC + lax.axis_index("c")
    base = wid * BPW
    pltpu.sync_copy(slot_hbm.at[pl.ds(base, BPW)], idx_v)
    pltpu.async_copy(y_hbm.at[idx_v], rows_v, sem).wait()
    pltpu.sync_copy(rows_v, out_hbm.at[pl.ds(base, BPW)])


def _sc_call(body, out_rows):
    mesh = plsc.VectorSubcoreMesh(core_axis_name="c", subcore_axis_name="s")
    return pl.kernel(
        body,
        out_type=jax.ShapeDtypeStruct((out_rows, C), jnp.float32),
        mesh=mesh,
        scratch_types=[
            pltpu.VMEM((BPW,), jnp.int32),
            pltpu.VMEM((BPW, C), jnp.float32),
            pltpu.SemaphoreType.DMA,
        ],
    )


RCH = 256


def _router_body(x_ref, wg_ref, bg_ref, slot_ref, texp_ref, tvalid_ref,
                 tclamp_ref, ranks_ref, oh_ref):
    scores = jnp.dot(x_ref[...].astype(jnp.bfloat16),
                     wg_ref[...].astype(jnp.bfloat16),
                     preferred_element_type=jnp.float32) + bg_ref[0, :][None, :]
    m = jnp.max(scores, axis=1, keepdims=True)
    iota_l = jax.lax.broadcasted_iota(jnp.int32, (N, E), 1)
    eid = jnp.min(jnp.where(scores == m, iota_l, E), axis=1, keepdims=True)
    onehot = (iota_l == eid).astype(jnp.float32)
    oh_ref[...] = onehot.astype(jnp.bfloat16)

    lt = (jax.lax.broadcasted_iota(jnp.int32, (RCH, RCH), 1)
          <= jax.lax.broadcasted_iota(jnp.int32, (RCH, RCH), 0))
    lt = lt.astype(jnp.bfloat16)

    def chunk(c, running):
        row0 = pl.multiple_of(c * RCH, RCH)
        oh_c = oh_ref[pl.ds(row0, RCH), :]
        local = jax.lax.dot_general(lt, oh_c, (((1,), (0,)), ((), ())),
                                    preferred_element_type=jnp.float32)
        ranks_ref[pl.ds(row0, RCH), :] = local + running
        return running + local[RCH - 1:RCH, :]

    counts = jax.lax.fori_loop(0, N // RCH, chunk,
                               jnp.zeros((1, E), jnp.float32))

    rank_own = jnp.sum(onehot * ranks_ref[...], axis=1, keepdims=True) - 1.0

    nt = jnp.floor((counts + (TILE - 1)) * (1.0 / TILE))
    ut = (jax.lax.broadcasted_iota(jnp.int32, (E, E), 0)
          < jax.lax.broadcasted_iota(jnp.int32, (E, E), 1)).astype(jnp.bfloat16)
    nt8 = jnp.broadcast_to(nt.astype(jnp.bfloat16), (8, E))
    ts = jax.lax.dot_general(nt8, ut, (((1,), (0,)), ((), ())),
                             preferred_element_type=jnp.float32)[0:1, :]
    t_act = jnp.sum(nt, axis=1, keepdims=True)

    slot = jnp.sum(onehot * ts, axis=1, keepdims=True) * TILE + rank_own
    slot_ref[...] = slot.astype(jnp.int32)

    t_row = jax.lax.broadcasted_iota(jnp.int32, (NT, E), 0).astype(jnp.float32)
    tsb = jnp.broadcast_to(ts, (NT, E))
    ntb = jnp.broadcast_to(nt, (NT, E))
    in_e = ((t_row >= tsb) & (t_row < tsb + ntb)).astype(jnp.float32)
    lane_e = jax.lax.broadcasted_iota(jnp.int32, (NT, E), 1).astype(jnp.float32)
    texp = jnp.sum(in_e * lane_e, axis=1, keepdims=True)
    t_col = jax.lax.broadcasted_iota(jnp.int32, (NT, 1), 0).astype(jnp.float32)
    t_act_b = jnp.broadcast_to(t_act, (NT, 1))
    valid = t_col < t_act_b
    last_e = jnp.sum(jnp.where(t_col == t_act_b - 1.0, texp, 0.0),
                     axis=0, keepdims=True)
    texp = jnp.where(valid, texp, jnp.broadcast_to(last_e, (NT, 1)))
    texp_ref[...] = texp.astype(jnp.int32)
    tvalid_ref[...] = valid.astype(jnp.int32)
    tclamp_ref[...] = jnp.minimum(t_col, t_act_b - 1.0).astype(jnp.int32)


_router_call = pl.pallas_call(
    _router_body,
    out_shape=[
        jax.ShapeDtypeStruct((N, 1), jnp.int32),
        jax.ShapeDtypeStruct((NT, 1), jnp.int32),
        jax.ShapeDtypeStruct((NT, 1), jnp.int32),
        jax.ShapeDtypeStruct((NT, 1), jnp.int32),
    ],
    scratch_shapes=[
        pltpu.VMEM((N, E), jnp.float32),
        pltpu.VMEM((N, E), jnp.bfloat16),
    ],
)


def _ffn_body(texp_ref, tvalid_ref, tclamp_ref, x_ref, w1_ref, b1_ref,
              w2_ref, b2_ref, out_ref, h_ref):
    t = pl.program_id(0)

    @pl.when(tvalid_ref[t, 0] != 0)
    def _():
        xb = x_ref[...].astype(jnp.bfloat16)
        xw = jnp.dot(xb, w1_ref[0].astype(jnp.bfloat16),
                     preferred_element_type=jnp.float32)
        h_ref[...] = jnp.maximum(xw + b1_ref[0, 0, :][None, :],
                                 0.0).astype(jnp.bfloat16)
        yk = jnp.dot(h_ref[...], w2_ref[0].astype(jnp.bfloat16),
                     preferred_element_type=jnp.float32)
        out_ref[...] = yk + b2_ref[0, 0, :][None, :]


_ffn_call = pl.pallas_call(
    _ffn_body,
    grid_spec=pltpu.PrefetchScalarGridSpec(
        num_scalar_prefetch=3,
        grid=(NT,),
        in_specs=[
            pl.BlockSpec((TILE, C), lambda t, te, tv, tc: (tc[t, 0], 0)),
            pl.BlockSpec((1, C, H), lambda t, te, tv, tc: (te[t, 0], 0, 0)),
            pl.BlockSpec((1, 1, H), lambda t, te, tv, tc: (te[t, 0], 0, 0)),
            pl.BlockSpec((1, H, C), lambda t, te, tv, tc: (te[t, 0], 0, 0)),
            pl.BlockSpec((1, 1, C), lambda t, te, tv, tc: (te[t, 0], 0, 0)),
        ],
        out_specs=pl.BlockSpec((TILE, C), lambda t, te, tv, tc: (tc[t, 0], 0)),
        scratch_shapes=[pltpu.VMEM((TILE, H), jnp.bfloat16)],
    ),
    out_shape=jax.ShapeDtypeStruct((NSLOT, C), jnp.float32),
)


def kernel(x, Wg, bg, W1, b1, W2, b2):
    Bv, Tv, _ = x.shape
    x_flat = x.reshape(Bv * Tv, C)

    slot2, texp, tile_valid, tclamp = _router_call(x_flat, Wg,
                                                   bg.reshape(1, E))
    slot = slot2.reshape(N)

    x_sorted = _sc_call(_scatter_body, NSLOT)(x_flat, slot)
    y_sorted = _ffn_call(texp, tile_valid, tclamp, x_sorted, W1,
                         b1.reshape(E, 1, H), W2, b2.reshape(E, 1, C))
    out = _sc_call(_gather_body, N)(y_sorted, slot)
    return out.reshape(Bv, Tv, C)
